# baseline (speedup 1.0000x reference)
import jax
import jax.numpy as jnp
import numpy as np
from jax.experimental import pallas as pl

_CAT_DIMS = [100] * 26
_N_CAT = len(_CAT_DIMS)
_N_CONT = 13
_CAT_W = sum(_CAT_DIMS)
_OUT_W = _CAT_W + _N_CONT
_BATCH = 16384
_BC = 512


def _build_tables():
    feat = np.zeros((_CAT_W,), dtype=np.int64)
    within = np.full((_OUT_W, 1), -1.0, dtype=np.float32)
    off = 0
    for i, w in enumerate(_CAT_DIMS):
        feat[off:off + w] = i
        within[off:off + w, 0] = np.arange(w, dtype=np.float32)
        off += w
    M_T = np.zeros((_OUT_W, _N_CAT), dtype=np.float32)
    M_T[np.arange(_CAT_W), feat] = 1.0
    P_T = np.zeros((_OUT_W, _N_CONT), dtype=np.float32)
    P_T[_CAT_W + np.arange(_N_CONT), np.arange(_N_CONT)] = 1.0
    return (jnp.asarray(M_T, dtype=jnp.bfloat16), jnp.asarray(within),
            jnp.asarray(P_T))


def _body(xc_ref, xt_ref, med_ref, fac_ref, m_ref, within_ref, p_ref, out_ref):
    xc = xc_ref[...].astype(jnp.bfloat16)
    g = jnp.dot(m_ref[...], xc, preferred_element_type=jnp.float32)
    oh = (g == within_ref[...]).astype(jnp.float32)

    xs = fac_ref[...] * (xt_ref[...] - med_ref[...])
    t = xs / jnp.sqrt(1.0 + (xs / 3.0) ** 2)
    out_ref[...] = oh + jnp.dot(p_ref[...], t, preferred_element_type=jnp.float32)


@jax.jit
def _run(x_cat_t, x_cont_t, median, factors, M_T, within, P_T):
    grid = (_BATCH // _BC,)
    out_t = pl.pallas_call(
        _body,
        grid=grid,
        in_specs=[
            pl.BlockSpec((_N_CAT, _BC), lambda i: (0, i)),
            pl.BlockSpec((_N_CONT, _BC), lambda i: (0, i)),
            pl.BlockSpec((_N_CONT, 1), lambda i: (0, 0)),
            pl.BlockSpec((_N_CONT, 1), lambda i: (0, 0)),
            pl.BlockSpec((_OUT_W, _N_CAT), lambda i: (0, 0)),
            pl.BlockSpec((_OUT_W, 1), lambda i: (0, 0)),
            pl.BlockSpec((_OUT_W, _N_CONT), lambda i: (0, 0)),
        ],
        out_specs=pl.BlockSpec((_OUT_W, _BC), lambda i: (0, i)),
        out_shape=jax.ShapeDtypeStruct((_OUT_W, _BATCH), jnp.float32),
    )(x_cat_t, x_cont_t, median.reshape(-1, 1), factors.reshape(-1, 1),
      M_T, within, P_T)
    return out_t.T


def kernel(x_cat, x_cont, median, factors):
    M_T, within, P_T = _build_tables()
    return _run(x_cat.astype(jnp.int32).T, x_cont.T, median, factors,
                M_T, within, P_T)


# speedup vs baseline: 1.0679x; 1.0679x over previous
import jax
import jax.numpy as jnp
import numpy as np
from jax.experimental import pallas as pl

_CAT_DIMS = [100] * 26
_N_CAT = len(_CAT_DIMS)
_N_CONT = 13
_CAT_W = sum(_CAT_DIMS)
_OUT_W = _CAT_W + _N_CONT
_BATCH = 16384
_BC = 1024


def _build_tables():
    feat = np.zeros((_CAT_W,), dtype=np.int64)
    within = np.full((_OUT_W, 1), -1.0, dtype=np.float32)
    off = 0
    for i, w in enumerate(_CAT_DIMS):
        feat[off:off + w] = i
        within[off:off + w, 0] = np.arange(w, dtype=np.float32)
        off += w
    M_T = np.zeros((_OUT_W, _N_CAT), dtype=np.float32)
    M_T[np.arange(_CAT_W), feat] = 1.0
    P_T = np.zeros((_OUT_W, _N_CONT), dtype=np.float32)
    P_T[_CAT_W + np.arange(_N_CONT), np.arange(_N_CONT)] = 1.0
    return (jnp.asarray(M_T, dtype=jnp.bfloat16), jnp.asarray(within),
            jnp.asarray(P_T))


def _body(xc_ref, xt_ref, med_ref, fac_ref, m_ref, within_ref, p_ref, out_ref):
    xc = xc_ref[...].astype(jnp.bfloat16)
    g = jnp.dot(m_ref[...], xc, preferred_element_type=jnp.float32)
    oh = (g == within_ref[...]).astype(jnp.float32)

    xs = fac_ref[...] * (xt_ref[...] - med_ref[...])
    t = xs / jnp.sqrt(1.0 + (xs / 3.0) ** 2)
    out_ref[...] = oh + jnp.dot(p_ref[...], t, preferred_element_type=jnp.float32)


@jax.jit
def _run(x_cat_t, x_cont_t, median, factors, M_T, within, P_T):
    grid = (_BATCH // _BC,)
    out_t = pl.pallas_call(
        _body,
        grid=grid,
        in_specs=[
            pl.BlockSpec((_N_CAT, _BC), lambda i: (0, i)),
            pl.BlockSpec((_N_CONT, _BC), lambda i: (0, i)),
            pl.BlockSpec((_N_CONT, 1), lambda i: (0, 0)),
            pl.BlockSpec((_N_CONT, 1), lambda i: (0, 0)),
            pl.BlockSpec((_OUT_W, _N_CAT), lambda i: (0, 0)),
            pl.BlockSpec((_OUT_W, 1), lambda i: (0, 0)),
            pl.BlockSpec((_OUT_W, _N_CONT), lambda i: (0, 0)),
        ],
        out_specs=pl.BlockSpec((_OUT_W, _BC), lambda i: (0, i)),
        out_shape=jax.ShapeDtypeStruct((_OUT_W, _BATCH), jnp.float32),
    )(x_cat_t, x_cont_t, median.reshape(-1, 1), factors.reshape(-1, 1),
      M_T, within, P_T)
    return out_t.T


def kernel(x_cat, x_cont, median, factors):
    M_T, within, P_T = _build_tables()
    return _run(x_cat.astype(jnp.int32).T, x_cont.T, median, factors,
                M_T, within, P_T)
